# Initial kernel scaffold; baseline (speedup 1.0000x reference)
import jax
import jax.numpy as jnp
from jax.experimental import pallas as pl


def kernel(cam):
    raise NotImplementedError("write your pallas kernel here")


# speedup vs baseline: 32.5631x; 32.5631x over previous
import functools

import jax
import jax.numpy as jnp
from jax.experimental import pallas as pl
from jax.experimental.pallas import tpu as pltpu

_PERCENT_T = 0.02
_H = 512
_W = 512
_N = _H * _W
_K = int(round(_N * _PERCENT_T))


def _row_kernel(x_ref, o_ref, key_ref):
    x = x_ref[0]
    b = jax.lax.bitcast_convert_type(x, jnp.int32)
    key = jnp.where(b < 0, b ^ jnp.int32(0x7FFFFFFF), b)
    key_ref[...] = key

    def body(i, prefix):
        bit = jnp.int32(31) - i
        cand = prefix ^ jnp.left_shift(jnp.int32(1), bit)
        cnt = jnp.sum((key_ref[...] >= cand).astype(jnp.int32))
        return jnp.where(cnt >= _K, cand, prefix)

    kth = jax.lax.fori_loop(0, 32, body, jnp.int32(-2147483648))

    keys = key_ref[...]
    gt = keys > kth
    cnt_gt = jnp.sum(gt.astype(jnp.int32))
    sum_gt = jnp.sum(jnp.where(gt, x, jnp.float32(0.0)))
    tb = jnp.where(kth < 0, kth ^ jnp.int32(0x7FFFFFFF), kth)
    t = jax.lax.bitcast_convert_type(tb, jnp.float32)
    mean = (sum_gt + (_K - cnt_gt).astype(jnp.float32) * t) / jnp.float32(_K)
    o_ref[pl.ds(pl.program_id(0), 1), :] = mean.reshape(1, 1)


@jax.jit
def kernel(cam):
    batch, ncls, h, w = cam.shape
    rows = cam.reshape(batch * ncls, h, w)
    out = pl.pallas_call(
        _row_kernel,
        grid=(batch * ncls,),
        in_specs=[pl.BlockSpec((1, h, w), lambda i: (i, 0, 0))],
        out_specs=pl.BlockSpec((batch * ncls, 1), lambda i: (0, 0)),
        out_shape=jax.ShapeDtypeStruct((batch * ncls, 1), jnp.float32),
        scratch_shapes=[pltpu.VMEM((h, w), jnp.int32)],
    )(rows)
    return out.reshape(batch, ncls)
